# baseline (speedup 1.0000x reference)
import functools

import jax
import jax.numpy as jnp
from jax.experimental import pallas as pl

K = 1024
D = 64
DECAY = 0.99
EPSILON = 1e-05


def _vq_body(x_ref, cb_ref, emac_ref, emaw_ref,
             disc_ref, quant_ref, cnt_ref, wgt_ref, emb_ref,
             *, nblocks, batch_size):
    i = pl.program_id(0)
    xb = x_ref[...]
    cb = cb_ref[...]

    x2 = jnp.sum(xb * xb, axis=1, keepdims=True)
    c2 = jnp.sum(cb * cb, axis=1, keepdims=True).T
    dot = jax.lax.dot_general(xb, cb, (((1,), (1,)), ((), ())),
                              preferred_element_type=jnp.float32)
    d = (x2 + (-2.0) * dot) + c2

    idx = jnp.argmin(d, axis=1)
    T = xb.shape[0]
    disc = (jax.lax.broadcasted_iota(jnp.int32, (T, K), 1)
            == idx[:, None]).astype(jnp.float32)
    disc_ref[...] = disc
    quant_ref[...] = jnp.dot(disc, cb, preferred_element_type=jnp.float32)

    ones_col = jnp.ones((T, 1), dtype=jnp.float32)
    cnt_blk = jax.lax.dot_general(disc, ones_col, (((0,), (0,)), ((), ())),
                                  preferred_element_type=jnp.float32)
    wgt_blk = jax.lax.dot_general(disc, xb, (((0,), (0,)), ((), ())),
                                  preferred_element_type=jnp.float32)

    @pl.when(i == 0)
    def _init():
        cnt_ref[...] = emac_ref[...] * DECAY
        wgt_ref[...] = emaw_ref[...] * DECAY

    cnt_ref[...] += (1.0 - DECAY) * cnt_blk
    wgt_ref[...] += (1.0 - DECAY) * wgt_blk

    @pl.when(i == nblocks - 1)
    def _finalize():
        nc = ((cnt_ref[...] + EPSILON)
              / (batch_size + K * EPSILON) * batch_size)
        cnt_ref[...] = nc
        emb_ref[...] = wgt_ref[...] / nc


def kernel(x, codebook, ema_count, ema_weight):
    batch_size = x.shape[0]
    x_flat = x.reshape(-1, D)
    N = x_flat.shape[0]
    T = 256
    nblocks = N // T

    body = functools.partial(_vq_body, nblocks=nblocks, batch_size=batch_size)
    disc, quant, new_count, new_weight, new_emb = pl.pallas_call(
        body,
        grid=(nblocks,),
        in_specs=[
            pl.BlockSpec((T, D), lambda i: (i, 0)),
            pl.BlockSpec((K, D), lambda i: (0, 0)),
            pl.BlockSpec((K, 1), lambda i: (0, 0)),
            pl.BlockSpec((K, D), lambda i: (0, 0)),
        ],
        out_specs=[
            pl.BlockSpec((T, K), lambda i: (i, 0)),
            pl.BlockSpec((T, D), lambda i: (i, 0)),
            pl.BlockSpec((K, 1), lambda i: (0, 0)),
            pl.BlockSpec((K, D), lambda i: (0, 0)),
            pl.BlockSpec((K, D), lambda i: (0, 0)),
        ],
        out_shape=[
            jax.ShapeDtypeStruct((N, K), jnp.float32),
            jax.ShapeDtypeStruct((N, D), jnp.float32),
            jax.ShapeDtypeStruct((K, 1), jnp.float32),
            jax.ShapeDtypeStruct((K, D), jnp.float32),
            jax.ShapeDtypeStruct((K, D), jnp.float32),
        ],
    )(x_flat, codebook, ema_count.reshape(K, 1), ema_weight)

    quantized = quant.reshape(x.shape)
    return (disc, quantized, new_count.reshape(K), new_weight, new_emb)


# speedup vs baseline: 1.3319x; 1.3319x over previous
import functools

import jax
import jax.numpy as jnp
from jax.experimental import pallas as pl

K = 1024
D = 64
DECAY = 0.99
EPSILON = 1e-05


def _vq_body(x_ref, cb_ref, disc_ref, quant_ref, acc_ref, *, nblocks):
    i = pl.program_id(0)
    xb = x_ref[...]
    cb = cb_ref[...]
    T = xb.shape[0]

    x2 = jnp.sum(xb * xb, axis=1, keepdims=True)
    c2 = jnp.sum(cb * cb, axis=1, keepdims=True).T
    dot = jax.lax.dot_general(xb, cb, (((1,), (1,)), ((), ())),
                              preferred_element_type=jnp.float32)
    d = (x2 + (-2.0) * dot) + c2

    idx = jnp.argmin(d, axis=1)
    disc = (jax.lax.broadcasted_iota(jnp.int32, (T, K), 1)
            == idx[:, None]).astype(jnp.float32)
    disc_ref[...] = disc
    quant_ref[...] = jnp.dot(disc, cb, preferred_element_type=jnp.float32)

    ext = (jax.lax.broadcasted_iota(jnp.int32, (T, D), 1)
           == 0).astype(jnp.float32)
    xb_ext = jnp.concatenate([xb, ext], axis=1)
    acc_blk = jax.lax.dot_general(disc, xb_ext, (((0,), (0,)), ((), ())),
                                  preferred_element_type=jnp.float32)

    @pl.when(i == 0)
    def _init():
        acc_ref[...] = jnp.zeros_like(acc_ref)

    acc_ref[...] += acc_blk


def _finalize_body(acc_ref, emac_ref, emaw_ref, cnt_ref, wgt_ref, emb_ref,
                   *, batch_size):
    acc = acc_ref[...]
    counts = acc[:, D:D + 1]
    sums = acc[:, :D]
    nc = emac_ref[...] * DECAY + counts * (1.0 - DECAY)
    nc = (nc + EPSILON) / (batch_size + K * EPSILON) * batch_size
    nw = emaw_ref[...] * DECAY + sums * (1.0 - DECAY)
    cnt_ref[...] = nc
    wgt_ref[...] = nw
    emb_ref[...] = nw / nc


def kernel(x, codebook, ema_count, ema_weight):
    batch_size = x.shape[0]
    x_flat = x.reshape(-1, D)
    N = x_flat.shape[0]
    T = 512
    nblocks = N // T

    body = functools.partial(_vq_body, nblocks=nblocks)
    disc, quant, acc = pl.pallas_call(
        body,
        grid=(nblocks,),
        in_specs=[
            pl.BlockSpec((T, D), lambda i: (i, 0)),
            pl.BlockSpec((K, D), lambda i: (0, 0)),
        ],
        out_specs=[
            pl.BlockSpec((T, K), lambda i: (i, 0)),
            pl.BlockSpec((T, D), lambda i: (i, 0)),
            pl.BlockSpec((K, 2 * D), lambda i: (0, 0)),
        ],
        out_shape=[
            jax.ShapeDtypeStruct((N, K), jnp.float32),
            jax.ShapeDtypeStruct((N, D), jnp.float32),
            jax.ShapeDtypeStruct((K, 2 * D), jnp.float32),
        ],
    )(x_flat, codebook)

    fin = functools.partial(_finalize_body, batch_size=batch_size)
    new_count, new_weight, new_emb = pl.pallas_call(
        fin,
        out_shape=[
            jax.ShapeDtypeStruct((K, 1), jnp.float32),
            jax.ShapeDtypeStruct((K, D), jnp.float32),
            jax.ShapeDtypeStruct((K, D), jnp.float32),
        ],
    )(acc, ema_count.reshape(K, 1), ema_weight)

    quantized = quant.reshape(x.shape)
    return (disc, quantized, new_count.reshape(K), new_weight, new_emb)
